# baseline (speedup 1.0000x reference)
import jax
import jax.numpy as jnp
from jax import lax
from jax.experimental import pallas as pl
from jax.experimental.pallas import tpu as pltpu
from jax.experimental.pallas import tpu_sc as plsc

B = 4096
L = 200
D = 64
C = 2
V = 1000000

P8 = 8

NC = 2
NS = 16
NW = NC * NS
BPW = B // NW
HALF = L // 2
ROWS_I = 2 * BPW
NBUF = 4


VR = 8000


def _proj_body(t_ref, w_ref, p_ref):
    proj = lax.dot_general(t_ref[...], w_ref[...], (((1,), (1,)), ((), ())),
                           preferred_element_type=jnp.float32)
    p_ref[...] = jnp.concatenate(
        [proj, jnp.zeros((VR, P8 - C), jnp.float32)], axis=1)


_project = pl.pallas_call(
    _proj_body,
    grid=(V // VR,),
    in_specs=[
        pl.BlockSpec((VR, D), lambda i: (i, 0)),
        pl.BlockSpec((C, D), lambda i: (0, 0)),
    ],
    out_specs=pl.BlockSpec((VR, P8), lambda i: (i, 0)),
    out_shape=jax.ShapeDtypeStruct((V, P8), jnp.float32),
)


def _sc_body(ids_hbm, p_hbm, out_hbm, ids_v, rows_v, out_v, s0, s1, s2, s3):
    sems = (s0, s1, s2, s3)
    wid = lax.axis_index("s") * NC + lax.axis_index("c")
    base = wid * BPW
    pltpu.sync_copy(ids_hbm.at[pl.ds(base * 2, ROWS_I)], ids_v)

    def gather(c, j):
        return pltpu.make_async_copy(
            p_hbm.at[ids_v.at[c]], rows_v.at[j], sems[j])

    for j in range(NBUF):
        gather(j, j).start()

    lane = lax.iota(jnp.int32, 16)
    row_off = jnp.where(lane >= P8, 1, 0)
    col_idx = lane & (P8 - 1)

    def accumulate(j, acc):
        buf = rows_v.at[j]
        def tok(k, acc):
            x = plsc.load_gather(buf, [2 * k + row_off, col_idx])
            return acc + x
        return lax.fori_loop(0, HALF // 2, tok, acc, unroll=5)

    def pair_body(bb, _):
        c0 = 4 * bb
        z = jnp.zeros((16,), jnp.float32)
        acc = z
        for j in range(NBUF):
            c = c0 + j
            gather(c, j).wait()
            acc = accumulate(j, acc)
            nxt = jnp.minimum(c + NBUF, ROWS_I - 1)
            gather(nxt, j).start()
            if j % 2 == 1:
                out_v[2 * bb + j // 2, pl.ds(0, 16)] = acc
                acc = z
        return 0

    lax.fori_loop(0, BPW // 2, pair_body, 0)
    for j in range(NBUF):
        gather(ROWS_I - 1, j).wait()
    pltpu.sync_copy(out_v, out_hbm.at[pl.ds(base, BPW)])


_SC_CACHE = {}


def _sc_gather_sum_fn():
    if "k" not in _SC_CACHE:
        _SC_CACHE["k"] = pl.kernel(
            _sc_body,
            out_type=jax.ShapeDtypeStruct((B, 16), jnp.float32),
            mesh=plsc.VectorSubcoreMesh(
                core_axis_name="c", subcore_axis_name="s",
                num_cores=NC, num_subcores=NS,
            ),
            scratch_types=[
                pltpu.VMEM((ROWS_I, HALF), jnp.int32),
                pltpu.VMEM((NBUF, HALF, P8), jnp.float32),
                pltpu.VMEM((BPW, 16), jnp.float32),
                pltpu.SemaphoreType.DMA,
                pltpu.SemaphoreType.DMA,
                pltpu.SemaphoreType.DMA,
                pltpu.SemaphoreType.DMA,
            ],
            compiler_params=pltpu.CompilerParams(
                use_tc_tiling_on_sc=False, needs_layout_passes=False),
        )
    return _SC_CACHE["k"]


BB = 512


def _head_body(ids_ref, sums_ref, b_ref, out_ref):
    ids = ids_ref[...]
    cnt = jnp.sum((ids != 0).astype(jnp.float32), axis=1, keepdims=True)
    s = sums_ref[:, 0:P8] + sums_ref[:, P8:16]
    out_ref[...] = s[:, 0:C] / (cnt + 1e-8) + b_ref[...]


_head = pl.pallas_call(
    _head_body,
    grid=(B // BB,),
    in_specs=[
        pl.BlockSpec((BB, L), lambda i: (i, 0)),
        pl.BlockSpec((BB, 16), lambda i: (i, 0)),
        pl.BlockSpec((1, C), lambda i: (0, 0)),
    ],
    out_specs=pl.BlockSpec((BB, C), lambda i: (i, 0)),
    out_shape=jax.ShapeDtypeStruct((B, C), jnp.float32),
)


def kernel(input_ids, table, W, b):
    ids = input_ids.astype(jnp.int32)
    p = _project(table, W)
    sums = _sc_gather_sum_fn()(ids.reshape(B * 2, HALF), p)
    return _head(ids, sums, b.reshape(1, C))


# speedup vs baseline: 1.0197x; 1.0197x over previous
import jax
import jax.numpy as jnp
from jax import lax
from jax.experimental import pallas as pl
from jax.experimental.pallas import tpu as pltpu
from jax.experimental.pallas import tpu_sc as plsc

B = 4096
L = 200
D = 64
C = 2
V = 1000000

NC = 2
NS = 16
NW = NC * NS
BPW = B // NW
CHUNK = 112
ROWS_I = 2 * BPW
NBUF = 4


BLKV = 8192


def _proj_body(t_ref, w_ref, p0_ref, p1_ref):
    d = lax.dot_general(w_ref[...], t_ref[...], (((1,), (1,)), ((), ())),
                        preferred_element_type=jnp.float32)
    p0_ref[...] = d[0, :]
    p1_ref[...] = d[1, :]


_project = pl.pallas_call(
    _proj_body,
    grid=(pl.cdiv(V, BLKV),),
    in_specs=[
        pl.BlockSpec((BLKV, D), lambda i: (i, 0)),
        pl.BlockSpec((C, D), lambda i: (0, 0)),
    ],
    out_specs=[
        pl.BlockSpec((BLKV,), lambda i: (i,)),
        pl.BlockSpec((BLKV,), lambda i: (i,)),
    ],
    out_shape=[
        jax.ShapeDtypeStruct((V,), jnp.float32),
        jax.ShapeDtypeStruct((V,), jnp.float32),
    ],
)


def _sc_body(ids_hbm, p0_hbm, p1_hbm, out_hbm, ids_v, rows_v, out_v,
             s0, s1, s2, s3):
    sems = (s0, s1, s2, s3)
    ps = (p0_hbm, p1_hbm)
    wid = lax.axis_index("s") * NC + lax.axis_index("c")
    base = wid * BPW
    pltpu.sync_copy(ids_hbm.at[pl.ds(base * 2, ROWS_I)], ids_v)

    z = jnp.zeros((16,), jnp.float32)

    def gather(c, j):
        copies = [
            pltpu.make_async_copy(
                ps[k].at[ids_v.at[c]], rows_v.at[j, k], sems[j])
            for k in range(C)
        ]
        return copies

    def start(c, j):
        for cp in gather(c, j):
            cp.start()

    def wait(c, j):
        for cp in gather(c, j):
            cp.wait()

    for j in range(NBUF):
        start(j, j)

    def accumulate(j, accs):
        out = []
        for k in range(C):
            acc = accs[k]
            def tok(t, acc, k=k):
                return acc + rows_v[j, k, pl.ds(16 * t, 16)]
            out.append(lax.fori_loop(0, CHUNK // 16, tok, acc, unroll=7))
        return tuple(out)

    def pair_body(bb, _):
        c0 = 4 * bb
        accs = (z, z)
        for j in range(NBUF):
            c = c0 + j
            wait(c, j)
            accs = accumulate(j, accs)
            nxt = jnp.minimum(c + NBUF, ROWS_I - 1)
            start(nxt, j)
            if j % 2 == 1:
                out_v[2 * bb + j // 2, pl.ds(0, 16)] = accs[0]
                out_v[2 * bb + j // 2, pl.ds(16, 16)] = accs[1]
                accs = (z, z)
        return 0

    lax.fori_loop(0, BPW // 2, pair_body, 0)
    for j in range(NBUF):
        wait(ROWS_I - 1, j)
    pltpu.sync_copy(out_v, out_hbm.at[pl.ds(base, BPW)])


_SC_CACHE = {}


def _sc_gather_sum_fn():
    if "k" not in _SC_CACHE:
        _SC_CACHE["k"] = pl.kernel(
            _sc_body,
            out_type=jax.ShapeDtypeStruct((B, 2 * 16), jnp.float32),
            mesh=plsc.VectorSubcoreMesh(
                core_axis_name="c", subcore_axis_name="s",
                num_cores=NC, num_subcores=NS,
            ),
            scratch_types=[
                pltpu.VMEM((ROWS_I, CHUNK), jnp.int32),
                pltpu.VMEM((NBUF, C, CHUNK), jnp.float32),
                pltpu.VMEM((BPW, 2 * 16), jnp.float32),
                pltpu.SemaphoreType.DMA,
                pltpu.SemaphoreType.DMA,
                pltpu.SemaphoreType.DMA,
                pltpu.SemaphoreType.DMA,
            ],
            compiler_params=pltpu.CompilerParams(
                use_tc_tiling_on_sc=False, needs_layout_passes=False),
        )
    return _SC_CACHE["k"]


BB = 512


def _head_body(ids_ref, sums_ref, b_ref, out_ref):
    ids = ids_ref[...]
    cnt = jnp.sum((ids != 0).astype(jnp.float32), axis=1, keepdims=True)
    s0 = jnp.sum(sums_ref[:, 0:16], axis=1, keepdims=True)
    s1 = jnp.sum(sums_ref[:, 16:32], axis=1, keepdims=True)
    s = jnp.concatenate([s0, s1], axis=1)
    out_ref[...] = s / (cnt + 1e-8) + b_ref[...]


_head = pl.pallas_call(
    _head_body,
    grid=(B // BB,),
    in_specs=[
        pl.BlockSpec((BB, L), lambda i: (i, 0)),
        pl.BlockSpec((BB, 2 * 16), lambda i: (i, 0)),
        pl.BlockSpec((1, C), lambda i: (0, 0)),
    ],
    out_specs=pl.BlockSpec((BB, C), lambda i: (i, 0)),
    out_shape=jax.ShapeDtypeStruct((B, C), jnp.float32),
)


def kernel(input_ids, table, W, b):
    ids = input_ids.astype(jnp.int32)
    p0, p1 = _project(table, W)
    ids_pad = jnp.pad(ids, ((0, 0), (0, 2 * CHUNK - L)))
    sums = _sc_gather_sum_fn()(ids_pad.reshape(B * 2, CHUNK), p0, p1)
    return _head(ids, sums, b.reshape(1, C))


# speedup vs baseline: 1.5356x; 1.5059x over previous
import jax
import jax.numpy as jnp
from jax import lax
from jax.experimental import pallas as pl
from jax.experimental.pallas import tpu as pltpu
from jax.experimental.pallas import tpu_sc as plsc

B = 4096
L = 200
D = 64
C = 2
V = 1000000
V8 = V // 8

NC = 2
NS = 16
NW = NC * NS
BPW = B // NW
CHUNK = 112
ROWS_I = 2 * BPW
NBUF = 4


BLKV = 8192


def _proj_body(t_ref, w_ref, p0_ref, p1_ref):
    d = lax.dot_general(w_ref[...], t_ref[...], (((1,), (0,)), ((), ())),
                        preferred_element_type=jnp.float32)
    p0_ref[...] = d[0, :]
    p1_ref[...] = d[1, :]


_project = pl.pallas_call(
    _proj_body,
    grid=(pl.cdiv(V, BLKV),),
    in_specs=[
        pl.BlockSpec((D, BLKV), lambda i: (0, i)),
        pl.BlockSpec((C, D), lambda i: (0, 0)),
    ],
    out_specs=[
        pl.BlockSpec((BLKV,), lambda i: (i,)),
        pl.BlockSpec((BLKV,), lambda i: (i,)),
    ],
    out_shape=[
        jax.ShapeDtypeStruct((V,), jnp.float32),
        jax.ShapeDtypeStruct((V,), jnp.float32),
    ],
)


def _sc_body(hi_hbm, lo_hbm, p0_hbm, p1_hbm, out_hbm,
             hi_v, lo_v, rows_v, out_v, s0, s1, s2, s3):
    sems = (s0, s1, s2, s3)
    ps = (p0_hbm, p1_hbm)
    wid = lax.axis_index("s") * NC + lax.axis_index("c")
    base = wid * BPW
    pltpu.sync_copy(hi_hbm.at[pl.ds(base * 2, ROWS_I)], hi_v)
    pltpu.sync_copy(lo_hbm.at[pl.ds(base * 2, ROWS_I)], lo_v)

    z = jnp.zeros((16,), jnp.float32)
    lane = lax.iota(jnp.int32, 16)

    def gather(c, j):
        return [
            pltpu.make_async_copy(
                ps[k].at[hi_v.at[c]], rows_v.at[2 * j + k], sems[j])
            for k in range(C)
        ]

    def start(c, j):
        for cp in gather(c, j):
            cp.start()

    def wait(c, j):
        for cp in gather(c, j):
            cp.wait()

    for j in range(NBUF):
        start(j, j)

    def accumulate(c, j, accs):
        def tok(t, accs):
            a0, a1 = accs
            col = lo_v[c, pl.ds(16 * t, 16)]
            row = 16 * t + lane
            a0 = a0 + plsc.load_gather(rows_v.at[2 * j + 0], [row, col])
            a1 = a1 + plsc.load_gather(rows_v.at[2 * j + 1], [row, col])
            return (a0, a1)
        return lax.fori_loop(0, CHUNK // 16, tok, accs, unroll=7)

    def pair_body(bb, _):
        c0 = 4 * bb
        accs = (z, z)
        for j in range(NBUF):
            c = c0 + j
            wait(c, j)
            accs = accumulate(c, j, accs)
            nxt = jnp.minimum(c + NBUF, ROWS_I - 1)
            start(nxt, j)
            if j % 2 == 1:
                out_v[2 * bb + j // 2, pl.ds(0, 16)] = accs[0]
                out_v[2 * bb + j // 2, pl.ds(16, 16)] = accs[1]
                accs = (z, z)
        return 0

    lax.fori_loop(0, BPW // 2, pair_body, 0)
    for j in range(NBUF):
        wait(ROWS_I - 1, j)
    pltpu.sync_copy(out_v, out_hbm.at[pl.ds(base, BPW)])


_SC_CACHE = {}


def _sc_gather_sum_fn():
    if "k" not in _SC_CACHE:
        _SC_CACHE["k"] = pl.kernel(
            _sc_body,
            out_type=jax.ShapeDtypeStruct((B, 2 * 16), jnp.float32),
            mesh=plsc.VectorSubcoreMesh(
                core_axis_name="c", subcore_axis_name="s",
                num_cores=NC, num_subcores=NS,
            ),
            scratch_types=[
                pltpu.VMEM((ROWS_I, CHUNK), jnp.int32),
                pltpu.VMEM((ROWS_I, CHUNK), jnp.int32),
                pltpu.VMEM((2 * NBUF, CHUNK, 8), jnp.float32),
                pltpu.VMEM((BPW, 2 * 16), jnp.float32),
                pltpu.SemaphoreType.DMA,
                pltpu.SemaphoreType.DMA,
                pltpu.SemaphoreType.DMA,
                pltpu.SemaphoreType.DMA,
            ],
            compiler_params=pltpu.CompilerParams(
                use_tc_tiling_on_sc=False, needs_layout_passes=False),
        )
    return _SC_CACHE["k"]


BB = 512


def _head_body(ids_ref, sums_ref, b_ref, out_ref):
    ids = ids_ref[...]
    cnt = jnp.sum((ids != 0).astype(jnp.float32), axis=1, keepdims=True)
    s0 = jnp.sum(sums_ref[:, 0:16], axis=1, keepdims=True)
    s1 = jnp.sum(sums_ref[:, 16:32], axis=1, keepdims=True)
    s = jnp.concatenate([s0, s1], axis=1)
    out_ref[...] = s / (cnt + 1e-8) + b_ref[...]


_head = pl.pallas_call(
    _head_body,
    grid=(B // BB,),
    in_specs=[
        pl.BlockSpec((BB, L), lambda i: (i, 0)),
        pl.BlockSpec((BB, 2 * 16), lambda i: (i, 0)),
        pl.BlockSpec((1, C), lambda i: (0, 0)),
    ],
    out_specs=pl.BlockSpec((BB, C), lambda i: (i, 0)),
    out_shape=jax.ShapeDtypeStruct((B, C), jnp.float32),
)


def kernel(input_ids, table, W, b):
    ids = input_ids.astype(jnp.int32)
    p0, p1 = _project(table.T, W)
    ids_pad = jnp.pad(ids, ((0, 0), (0, 2 * CHUNK - L))).reshape(B * 2, CHUNK)
    sums = _sc_gather_sum_fn()(
        ids_pad >> 3, ids_pad & 7,
        p0.reshape(V8, 8), p1.reshape(V8, 8))
    return _head(ids, sums, b.reshape(1, C))


# speedup vs baseline: 1.5836x; 1.0313x over previous
import jax
import jax.numpy as jnp
from jax import lax
from jax.experimental import pallas as pl
from jax.experimental.pallas import tpu as pltpu
from jax.experimental.pallas import tpu_sc as plsc

B = 4096
L = 200
D = 64
C = 2
V = 1000000
V8 = V // 8

NC = 2
NS = 16
NW = NC * NS
BPW = B // NW
CHUNK = 112
ROWS_I = 2 * BPW
NBUF = 4


BLKV = 8192


def _proj_body(t_ref, w_ref, q_ref):
    d = lax.dot_general(w_ref[...], t_ref[...], (((1,), (0,)), ((), ())),
                        preferred_element_type=jnp.float32)
    u0 = lax.bitcast_convert_type(
        d[0, :].astype(jnp.bfloat16), jnp.uint16).astype(jnp.uint32)
    u1 = lax.bitcast_convert_type(
        d[1, :].astype(jnp.bfloat16), jnp.uint16).astype(jnp.uint32)
    q_ref[...] = lax.bitcast_convert_type((u0 << 16) | u1, jnp.float32)


_project = pl.pallas_call(
    _proj_body,
    grid=(pl.cdiv(V, BLKV),),
    in_specs=[
        pl.BlockSpec((D, BLKV), lambda i: (0, i)),
        pl.BlockSpec((C, D), lambda i: (0, 0)),
    ],
    out_specs=pl.BlockSpec((BLKV,), lambda i: (i,)),
    out_shape=jax.ShapeDtypeStruct((V,), jnp.float32),
)


def _sc_body(hi_hbm, lo_hbm, q_hbm, out_hbm,
             hi_v, lo_v, rows_v, out_v, s0, s1, s2, s3):
    sems = (s0, s1, s2, s3)
    wid = lax.axis_index("s") * NC + lax.axis_index("c")
    base = wid * BPW
    pltpu.sync_copy(hi_hbm.at[pl.ds(base * 2, ROWS_I)], hi_v)
    pltpu.sync_copy(lo_hbm.at[pl.ds(base * 2, ROWS_I)], lo_v)

    z = jnp.zeros((16,), jnp.float32)
    lane = lax.iota(jnp.int32, 16)

    def gather(c, j):
        return pltpu.make_async_copy(
            q_hbm.at[hi_v.at[c]], rows_v.at[j], sems[j])

    def start(c, j):
        gather(c, j).start()

    def wait(c, j):
        gather(c, j).wait()

    for j in range(NBUF):
        start(j, j)

    hi_mask = jnp.full((16,), 0xFFFF0000, jnp.uint32)

    def accumulate(c, j, accs):
        def tok(t, accs):
            a0, a1 = accs
            col = lo_v[c, pl.ds(16 * t, 16)]
            row = 16 * t + lane
            x = plsc.bitcast(
                plsc.load_gather(rows_v.at[j], [row, col]), jnp.uint32)
            a0 = a0 + plsc.bitcast(x & hi_mask, jnp.float32)
            a1 = a1 + plsc.bitcast(x << 16, jnp.float32)
            return (a0, a1)
        return lax.fori_loop(0, CHUNK // 16, tok, accs, unroll=7)

    def pair_body(bb, _):
        c0 = 4 * bb
        accs = (z, z)
        for j in range(NBUF):
            c = c0 + j
            wait(c, j)
            accs = accumulate(c, j, accs)
            nxt = jnp.minimum(c + NBUF, ROWS_I - 1)
            start(nxt, j)
            if j % 2 == 1:
                out_v[2 * bb + j // 2, pl.ds(0, 16)] = accs[0]
                out_v[2 * bb + j // 2, pl.ds(16, 16)] = accs[1]
                accs = (z, z)
        return 0

    lax.fori_loop(0, BPW // 2, pair_body, 0)
    for j in range(NBUF):
        wait(ROWS_I - 1, j)
    pltpu.sync_copy(out_v, out_hbm.at[pl.ds(base, BPW)])


_SC_CACHE = {}


def _sc_gather_sum_fn():
    if "k" not in _SC_CACHE:
        _SC_CACHE["k"] = pl.kernel(
            _sc_body,
            out_type=jax.ShapeDtypeStruct((B, 2 * 16), jnp.float32),
            mesh=plsc.VectorSubcoreMesh(
                core_axis_name="c", subcore_axis_name="s",
                num_cores=NC, num_subcores=NS,
            ),
            scratch_types=[
                pltpu.VMEM((ROWS_I, CHUNK), jnp.int32),
                pltpu.VMEM((ROWS_I, CHUNK), jnp.int32),
                pltpu.VMEM((NBUF, CHUNK, 8), jnp.float32),
                pltpu.VMEM((BPW, 2 * 16), jnp.float32),
                pltpu.SemaphoreType.DMA,
                pltpu.SemaphoreType.DMA,
                pltpu.SemaphoreType.DMA,
                pltpu.SemaphoreType.DMA,
            ],
            compiler_params=pltpu.CompilerParams(
                use_tc_tiling_on_sc=False, needs_layout_passes=False),
        )
    return _SC_CACHE["k"]


BB = 512


def _head_body(ids_ref, sums_ref, b_ref, out_ref):
    ids = ids_ref[...]
    cnt = jnp.sum((ids != 0).astype(jnp.float32), axis=1, keepdims=True)
    s0 = jnp.sum(sums_ref[:, 0:16], axis=1, keepdims=True)
    s1 = jnp.sum(sums_ref[:, 16:32], axis=1, keepdims=True)
    s = jnp.concatenate([s0, s1], axis=1)
    out_ref[...] = s / (cnt + 1e-8) + b_ref[...]


_head = pl.pallas_call(
    _head_body,
    grid=(B // BB,),
    in_specs=[
        pl.BlockSpec((BB, L), lambda i: (i, 0)),
        pl.BlockSpec((BB, 2 * 16), lambda i: (i, 0)),
        pl.BlockSpec((1, C), lambda i: (0, 0)),
    ],
    out_specs=pl.BlockSpec((BB, C), lambda i: (i, 0)),
    out_shape=jax.ShapeDtypeStruct((B, C), jnp.float32),
)


def kernel(input_ids, table, W, b):
    ids = input_ids.astype(jnp.int32)
    q = _project(table.T, W)
    ids_pad = jnp.pad(ids, ((0, 0), (0, 2 * CHUNK - L))).reshape(B * 2, CHUNK)
    sums = _sc_gather_sum_fn()(ids_pad >> 3, ids_pad & 7, q.reshape(V8, 8))
    return _head(ids, sums, b.reshape(1, C))
